# baseline (speedup 1.0000x reference)
import functools

import jax
import jax.numpy as jnp
from jax import lax
from jax.experimental import pallas as pl
from jax.experimental.pallas import tpu as pltpu
from jax.experimental.pallas import tpu_sc as plsc

N = 10000
NPAD = 10240
E = 320000
K = 128
NC = 2
NS = 16
W_TOT = 2560
E_PAD = W_TOT * K
WPT = W_TOT // (NC * NS)
NFEAT = 128
NHID = 256
NCLS = 16
DEG_T = 10240
RPT = NPAD // NS

_MESH = dict(core_axis_name="c", subcore_axis_name="s")


@functools.partial(
    pl.kernel,
    out_type=jax.ShapeDtypeStruct((4 * N,), jnp.float32),
    mesh=plsc.VectorSubcoreMesh(**_MESH),
    compiler_params=pltpu.CompilerParams(use_tc_tiling_on_sc=False),
    scratch_types=[
        pltpu.VMEM((WPT, K), jnp.int32),
        pltpu.VMEM((WPT, K), jnp.int32),
        pltpu.VMEM((K,), jnp.float32),
        pltpu.VMEM((640,), jnp.float32),
        pltpu.VMEM((1000,), jnp.float32),
        pltpu.VMEM_SHARED((DEG_T,), jnp.float32),
        pltpu.VMEM_SHARED((DEG_T,), jnp.float32),
        pltpu.SemaphoreType.DMA,
    ],
)
def _deg_kernel(src_hbm, dst_hbm, out_hbm, src_v, dst_v, ones_v, z_v,
                stage_v, odeg_sh, ideg_sh, dsem):
    c = lax.axis_index("c")
    s = lax.axis_index("s")
    wid = c * NS + s

    def fill_ones(i, _):
        ones_v[pl.ds(i * 16, 16)] = jnp.full((16,), 1.0, jnp.float32)
        return 0

    lax.fori_loop(0, K // 16, fill_ones, 0)

    def fill_z(i, _):
        z_v[pl.ds(i * 16, 16)] = jnp.zeros((16,), jnp.float32)
        return 0

    lax.fori_loop(0, 640 // 16, fill_z, 0)

    pltpu.sync_copy(z_v, odeg_sh.at[pl.ds(s * 640, 640)])
    pltpu.sync_copy(z_v, ideg_sh.at[pl.ds(s * 640, 640)])
    pltpu.sync_copy(src_hbm.at[pl.ds(wid * WPT, WPT)], src_v)
    pltpu.sync_copy(dst_hbm.at[pl.ds(wid * WPT, WPT)], dst_v)
    plsc.subcore_barrier()

    def win(i, _):
        pltpu.async_copy(ones_v, odeg_sh.at[src_v.at[i]], dsem, add=True)
        pltpu.async_copy(ones_v, ideg_sh.at[dst_v.at[i]], dsem, add=True)
        return 0

    lax.fori_loop(0, WPT, win, 0)

    def drain(i, _):
        pltpu.make_async_copy(ones_v, odeg_sh.at[src_v.at[i]], dsem).wait()
        pltpu.make_async_copy(ones_v, ideg_sh.at[dst_v.at[i]], dsem).wait()
        return 0

    lax.fori_loop(0, WPT, drain, 0)
    plsc.subcore_barrier()

    @pl.when(s < 10)
    def _():
        base = s * 1000
        pltpu.sync_copy(odeg_sh.at[pl.ds(base, 1000)], stage_v)
        pltpu.sync_copy(stage_v,
                        out_hbm.at[pl.ds((2 * c + 0) * N + base, 1000)])
        pltpu.sync_copy(ideg_sh.at[pl.ds(base, 1000)], stage_v)
        pltpu.sync_copy(stage_v,
                        out_hbm.at[pl.ds((2 * c + 1) * N + base, 1000)])


RING = 4
PREF = 2


def _ring_pipeline(tab_hbm, src_v, dst_v, rows_v, acc_sh, gsem, ssem, nwin,
                   ring, pref, w0=0):
    for k in range(pref):
        pltpu.async_copy(tab_hbm.at[src_v.at[w0 + k]], rows_v.at[k],
                         gsem.at[k])

    def win(j, _):
        i = w0 + j
        bn = lax.rem(j + pref, ring)

        @pl.when(j + pref < nwin)
        def _():
            @pl.when(j >= ring - pref)
            def _():
                pltpu.make_async_copy(rows_v.at[bn],
                                      acc_sh.at[dst_v.at[i - ring + pref]],
                                      ssem.at[bn]).wait()

            pltpu.async_copy(tab_hbm.at[src_v.at[i + pref]], rows_v.at[bn],
                             gsem.at[bn])

        b = lax.rem(j, ring)
        pltpu.make_async_copy(tab_hbm.at[src_v.at[i]], rows_v.at[b],
                              gsem.at[b]).wait()
        pltpu.async_copy(rows_v.at[b], acc_sh.at[dst_v.at[i]],
                         ssem.at[b], add=True)
        return 0

    lax.fori_loop(0, nwin, win, 0)
    for k in range(ring):
        w = w0 + nwin - ring + k
        pltpu.make_async_copy(rows_v.at[(w - w0) % ring],
                              acc_sh.at[dst_v.at[w]],
                              ssem.at[(w - w0) % ring]).wait()


def _zero_acc(rows_v, acc_sh, s, D):
    def zrow(i, _):
        for j in range(D // 16):
            rows_v[0, i, pl.ds(j * 16, 16)] = jnp.zeros((16,), jnp.float32)
        return 0

    lax.fori_loop(0, K, zrow, 0)
    for r in range(RPT // K):
        pltpu.sync_copy(rows_v.at[0], acc_sh.at[pl.ds(s * RPT + r * K, K)])


DHALF = NFEAT // 2
WPS = W_TOT // NS


@functools.partial(
    pl.kernel,
    out_type=jax.ShapeDtypeStruct((2 * NPAD, DHALF), jnp.float32),
    mesh=plsc.VectorSubcoreMesh(**_MESH),
    compiler_params=pltpu.CompilerParams(use_tc_tiling_on_sc=False),
    scratch_types=[
        pltpu.VMEM((WPS, K), jnp.int32),
        pltpu.VMEM((WPS, K), jnp.int32),
        pltpu.VMEM((RING, K, DHALF), jnp.float32),
        pltpu.VMEM_SHARED((NPAD, DHALF), jnp.float32),
        pltpu.SemaphoreType.DMA((RING,)),
        pltpu.SemaphoreType.DMA((RING,)),
    ],
)
def _agg128(tab_hbm, srca_hbm, srcb_hbm, dst_hbm, out_hbm,
            src_v, dst_v, rows_v, acc_sh, gsem, ssem):
    c = lax.axis_index("c")
    s = lax.axis_index("s")

    _zero_acc(rows_v, acc_sh, s, DHALF)

    @pl.when(c == 0)
    def _():
        pltpu.sync_copy(srca_hbm.at[pl.ds(s * WPS, WPS)], src_v)

    @pl.when(c == 1)
    def _():
        pltpu.sync_copy(srcb_hbm.at[pl.ds(s * WPS, WPS)], src_v)

    pltpu.sync_copy(dst_hbm.at[pl.ds(s * WPS, WPS)], dst_v)
    plsc.subcore_barrier()

    _ring_pipeline(tab_hbm, src_v, dst_v, rows_v, acc_sh, gsem, ssem, WPS,
                   ring=RING, pref=PREF)

    plsc.subcore_barrier()
    for r in range(RPT // K):
        pltpu.sync_copy(acc_sh.at[pl.ds(s * RPT + r * K, K)], rows_v.at[0])
        pltpu.sync_copy(rows_v.at[0],
                        out_hbm.at[pl.ds(c * NPAD + s * RPT + r * K, K)])


@functools.partial(
    pl.kernel,
    out_type=jax.ShapeDtypeStruct((2 * NPAD, NCLS), jnp.float32),
    mesh=plsc.VectorSubcoreMesh(**_MESH),
    compiler_params=pltpu.CompilerParams(use_tc_tiling_on_sc=False),
    scratch_types=[
        pltpu.VMEM((WPT, K), jnp.int32),
        pltpu.VMEM((WPT, K), jnp.int32),
        pltpu.VMEM((RING, K, NCLS), jnp.float32),
        pltpu.VMEM_SHARED((NPAD, NCLS), jnp.float32),
        pltpu.SemaphoreType.DMA((RING,)),
        pltpu.SemaphoreType.DMA((RING,)),
    ],
)
def _agg16(tab_hbm, src_hbm, dst_hbm, out_hbm,
           src_v, dst_v, rows_v, acc_sh, gsem, ssem):
    c = lax.axis_index("c")
    s = lax.axis_index("s")
    wid = c * NS + s

    _zero_acc(rows_v, acc_sh, s, NCLS)
    pltpu.sync_copy(src_hbm.at[pl.ds(wid * WPT, WPT)], src_v)
    pltpu.sync_copy(dst_hbm.at[pl.ds(wid * WPT, WPT)], dst_v)
    plsc.subcore_barrier()

    _ring_pipeline(tab_hbm, src_v, dst_v, rows_v, acc_sh, gsem, ssem, WPT,
                   ring=RING, pref=PREF)

    plsc.subcore_barrier()
    for r in range(RPT // K):
        pltpu.sync_copy(acc_sh.at[pl.ds(s * RPT + r * K, K)], rows_v.at[0])
        pltpu.sync_copy(rows_v.at[0],
                        out_hbm.at[pl.ds(c * NPAD + s * RPT + r * K, K)])


def _prep_body(x_ref, degt_ref, out_ref):
    od = degt_ref[:, 0:1] + degt_ref[:, 2:3]
    ns = lax.rsqrt(jnp.maximum(od, 1.0))
    xn = x_ref[...] * ns
    zpad = jnp.zeros((NPAD - N, DHALF), jnp.float32)
    out_ref[0:N, :] = xn[:, 0:DHALF]
    out_ref[N:NPAD, :] = zpad
    out_ref[NPAD:NPAD + N, :] = xn[:, DHALF:NFEAT]
    out_ref[NPAD + N:2 * NPAD, :] = zpad


def _mid_body(aggp_ref, degt_ref, w1_ref, b1_ref, w2_ref, wfc_ref, out_ref):
    agg1 = jnp.concatenate(
        [aggp_ref[0:N, :], aggp_ref[NPAD:NPAD + N, :]], axis=1)
    ind = degt_ref[:, 1:2] + degt_ref[:, 3:4]
    od = degt_ref[:, 0:1] + degt_ref[:, 2:3]
    nd = lax.rsqrt(jnp.maximum(ind, 1.0))
    ns = lax.rsqrt(jnp.maximum(od, 1.0))
    h1 = jnp.dot(agg1 * nd, w1_ref[...], preferred_element_type=jnp.float32)
    h1 = jnp.maximum(h1 + b1_ref[...], 0.0)
    w2f = jnp.dot(w2_ref[...], wfc_ref[...], preferred_element_type=jnp.float32)
    g = jnp.dot(h1 * ns, w2f, preferred_element_type=jnp.float32)
    out_ref[0:N, :] = g
    out_ref[N:NPAD, :] = jnp.zeros((NPAD - N, NCLS), jnp.float32)


def _fin_body(agg2_ref, degt_ref, b2_ref, wfc_ref, bfc_ref, out_ref):
    agg2 = agg2_ref[0:N, :] + agg2_ref[NPAD:NPAD + N, :]
    ind = degt_ref[:, 1:2] + degt_ref[:, 3:4]
    nd = lax.rsqrt(jnp.maximum(ind, 1.0))
    bf = jnp.dot(b2_ref[...].reshape(1, NHID), wfc_ref[...],
                 preferred_element_type=jnp.float32)
    out_ref[...] = agg2 * nd + bf + bfc_ref[...].reshape(1, NCLS)


def kernel(edge_index, x, W1, b1, W2, b2, Wfc, bfc):
    ei = edge_index.astype(jnp.int32)
    pad = (jnp.arange(E_PAD - E, dtype=jnp.int32) % (NPAD - N)) + N
    src = jnp.concatenate([ei[0], pad]).reshape(W_TOT, K)
    dst = jnp.concatenate([ei[1], pad]).reshape(W_TOT, K)

    degs = _deg_kernel(src, dst)
    degt = jnp.transpose(degs.reshape(4, N))

    h = pl.pallas_call(
        _prep_body,
        out_shape=jax.ShapeDtypeStruct((2 * NPAD, DHALF), jnp.float32),
    )(x, degt)

    srcb = src + NPAD
    aggp = _agg128(h, src, srcb, dst)

    g = pl.pallas_call(
        _mid_body,
        out_shape=jax.ShapeDtypeStruct((NPAD, NCLS), jnp.float32),
    )(aggp, degt, W1, b1, W2, Wfc)

    agg2 = _agg16(g, src, dst)

    out = pl.pallas_call(
        _fin_body,
        out_shape=jax.ShapeDtypeStruct((N, NCLS), jnp.float32),
    )(agg2, degt, b2, Wfc, bfc)
    return out


# speedup vs baseline: 1.0609x; 1.0609x over previous
import functools

import jax
import jax.numpy as jnp
from jax import lax
from jax.experimental import pallas as pl
from jax.experimental.pallas import tpu as pltpu
from jax.experimental.pallas import tpu_sc as plsc

N = 10000
NPAD = 10240
E = 320000
K = 128
NC = 2
NS = 16
W_TOT = 2560
E_PAD = W_TOT * K
WPT = W_TOT // (NC * NS)
NFEAT = 128
NHID = 256
NCLS = 16
DEG_T = 10240
RPT = NPAD // NS

_MESH = dict(core_axis_name="c", subcore_axis_name="s")


@functools.partial(
    pl.kernel,
    out_type=jax.ShapeDtypeStruct((4 * N,), jnp.float32),
    mesh=plsc.VectorSubcoreMesh(**_MESH),
    compiler_params=pltpu.CompilerParams(use_tc_tiling_on_sc=False),
    scratch_types=[
        pltpu.VMEM((WPT, K), jnp.int32),
        pltpu.VMEM((WPT, K), jnp.int32),
        pltpu.VMEM((K,), jnp.float32),
        pltpu.VMEM((640,), jnp.float32),
        pltpu.VMEM((1000,), jnp.float32),
        pltpu.VMEM_SHARED((DEG_T,), jnp.float32),
        pltpu.VMEM_SHARED((DEG_T,), jnp.float32),
        pltpu.SemaphoreType.DMA,
    ],
)
def _deg_kernel(src_hbm, dst_hbm, out_hbm, src_v, dst_v, ones_v, z_v,
                stage_v, odeg_sh, ideg_sh, dsem):
    c = lax.axis_index("c")
    s = lax.axis_index("s")
    wid = c * NS + s

    def fill_ones(i, _):
        ones_v[pl.ds(i * 16, 16)] = jnp.full((16,), 1.0, jnp.float32)
        return 0

    lax.fori_loop(0, K // 16, fill_ones, 0)

    def fill_z(i, _):
        z_v[pl.ds(i * 16, 16)] = jnp.zeros((16,), jnp.float32)
        return 0

    lax.fori_loop(0, 640 // 16, fill_z, 0)

    pltpu.sync_copy(z_v, odeg_sh.at[pl.ds(s * 640, 640)])
    pltpu.sync_copy(z_v, ideg_sh.at[pl.ds(s * 640, 640)])
    pltpu.sync_copy(src_hbm.at[pl.ds(wid * WPT, WPT)], src_v)
    pltpu.sync_copy(dst_hbm.at[pl.ds(wid * WPT, WPT)], dst_v)
    plsc.subcore_barrier()

    def win(i, _):
        pltpu.async_copy(ones_v, odeg_sh.at[src_v.at[i]], dsem, add=True)
        pltpu.async_copy(ones_v, ideg_sh.at[dst_v.at[i]], dsem, add=True)
        return 0

    lax.fori_loop(0, WPT, win, 0)

    def drain(i, _):
        pltpu.make_async_copy(ones_v, odeg_sh.at[src_v.at[i]], dsem).wait()
        pltpu.make_async_copy(ones_v, ideg_sh.at[dst_v.at[i]], dsem).wait()
        return 0

    lax.fori_loop(0, WPT, drain, 0)
    plsc.subcore_barrier()

    @pl.when(s < 10)
    def _():
        base = s * 1000
        pltpu.sync_copy(odeg_sh.at[pl.ds(base, 1000)], stage_v)
        pltpu.sync_copy(stage_v,
                        out_hbm.at[pl.ds((2 * c + 0) * N + base, 1000)])
        pltpu.sync_copy(ideg_sh.at[pl.ds(base, 1000)], stage_v)
        pltpu.sync_copy(stage_v,
                        out_hbm.at[pl.ds((2 * c + 1) * N + base, 1000)])


RING = 4
PREF = 2


def _ring_pipeline(tab_hbm, src_v, dst_v, rows_v, acc_sh, gsem, ssem, nwin,
                   ring, pref, w0=0):
    for k in range(pref):
        pltpu.async_copy(tab_hbm.at[src_v.at[w0 + k]], rows_v.at[k],
                         gsem.at[k])

    def win(j, _):
        i = w0 + j
        bn = lax.rem(j + pref, ring)

        @pl.when(j + pref < nwin)
        def _():
            @pl.when(j >= ring - pref)
            def _():
                pltpu.make_async_copy(rows_v.at[bn],
                                      acc_sh.at[dst_v.at[i - ring + pref]],
                                      ssem.at[bn]).wait()

            pltpu.async_copy(tab_hbm.at[src_v.at[i + pref]], rows_v.at[bn],
                             gsem.at[bn])

        b = lax.rem(j, ring)
        pltpu.make_async_copy(tab_hbm.at[src_v.at[i]], rows_v.at[b],
                              gsem.at[b]).wait()
        pltpu.async_copy(rows_v.at[b], acc_sh.at[dst_v.at[i]],
                         ssem.at[b], add=True)
        return 0

    lax.fori_loop(0, nwin, win, 0)
    for k in range(ring):
        w = w0 + nwin - ring + k
        pltpu.make_async_copy(rows_v.at[(w - w0) % ring],
                              acc_sh.at[dst_v.at[w]],
                              ssem.at[(w - w0) % ring]).wait()


def _zero_acc(rows_v, acc_sh, s, D):
    def zrow(i, _):
        for j in range(D // 16):
            rows_v[0, i, pl.ds(j * 16, 16)] = jnp.zeros((16,), jnp.float32)
        return 0

    lax.fori_loop(0, K, zrow, 0)
    for r in range(RPT // K):
        pltpu.sync_copy(rows_v.at[0], acc_sh.at[pl.ds(s * RPT + r * K, K)])


DHALF = NFEAT // 2
WPH = WPT // 2


@functools.partial(
    pl.kernel,
    out_type=jax.ShapeDtypeStruct((2 * NPAD, NFEAT), jnp.float32),
    mesh=plsc.VectorSubcoreMesh(**_MESH),
    scratch_types=[
        pltpu.VMEM((WPH, K), jnp.int32),
        pltpu.VMEM((WPH, K), jnp.int32),
        pltpu.VMEM((2, K, NFEAT), jnp.float32),
        pltpu.VMEM_SHARED((NPAD, NFEAT), jnp.float32),
        pltpu.SemaphoreType.DMA((2,)),
        pltpu.SemaphoreType.DMA((2,)),
    ],
)
def _agg128(tab_hbm, src_hbm, dst_hbm, out_hbm,
            src_v, dst_v, rows_v, acc_sh, gsem, ssem):
    c = lax.axis_index("c")
    s = lax.axis_index("s")
    wid = c * NS + s

    _zero_acc(rows_v, acc_sh, s, NFEAT)
    plsc.subcore_barrier()

    for ph in range(2):
        base_w = wid * WPT + ph * WPH
        pltpu.sync_copy(src_hbm.at[pl.ds(base_w, WPH)], src_v)
        pltpu.sync_copy(dst_hbm.at[pl.ds(base_w, WPH)], dst_v)
        _ring_pipeline(tab_hbm, src_v, dst_v, rows_v, acc_sh, gsem, ssem,
                       WPH, ring=2, pref=1)

    plsc.subcore_barrier()
    for r in range(RPT // K):
        pltpu.sync_copy(acc_sh.at[pl.ds(s * RPT + r * K, K)], rows_v.at[0])
        pltpu.sync_copy(rows_v.at[0],
                        out_hbm.at[pl.ds(c * NPAD + s * RPT + r * K, K)])


@functools.partial(
    pl.kernel,
    out_type=jax.ShapeDtypeStruct((2 * NPAD, NCLS), jnp.float32),
    mesh=plsc.VectorSubcoreMesh(**_MESH),
    compiler_params=pltpu.CompilerParams(use_tc_tiling_on_sc=False),
    scratch_types=[
        pltpu.VMEM((WPT, K), jnp.int32),
        pltpu.VMEM((WPT, K), jnp.int32),
        pltpu.VMEM((RING, K, NCLS), jnp.float32),
        pltpu.VMEM_SHARED((NPAD, NCLS), jnp.float32),
        pltpu.SemaphoreType.DMA((RING,)),
        pltpu.SemaphoreType.DMA((RING,)),
    ],
)
def _agg16(tab_hbm, src_hbm, dst_hbm, out_hbm,
           src_v, dst_v, rows_v, acc_sh, gsem, ssem):
    c = lax.axis_index("c")
    s = lax.axis_index("s")
    wid = c * NS + s

    _zero_acc(rows_v, acc_sh, s, NCLS)
    pltpu.sync_copy(src_hbm.at[pl.ds(wid * WPT, WPT)], src_v)
    pltpu.sync_copy(dst_hbm.at[pl.ds(wid * WPT, WPT)], dst_v)
    plsc.subcore_barrier()

    _ring_pipeline(tab_hbm, src_v, dst_v, rows_v, acc_sh, gsem, ssem, WPT,
                   ring=RING, pref=PREF)

    plsc.subcore_barrier()
    for r in range(RPT // K):
        pltpu.sync_copy(acc_sh.at[pl.ds(s * RPT + r * K, K)], rows_v.at[0])
        pltpu.sync_copy(rows_v.at[0],
                        out_hbm.at[pl.ds(c * NPAD + s * RPT + r * K, K)])


def _prep_body(x_ref, degt_ref, out_ref):
    od = degt_ref[:, 0:1] + degt_ref[:, 2:3]
    ns = lax.rsqrt(jnp.maximum(od, 1.0))
    out_ref[0:N, :] = x_ref[...] * ns
    out_ref[N:NPAD, :] = jnp.zeros((NPAD - N, NFEAT), jnp.float32)


def _mid_body(aggp_ref, degt_ref, w1_ref, b1_ref, w2_ref, wfc_ref, out_ref):
    agg1 = aggp_ref[0:N, :] + aggp_ref[NPAD:NPAD + N, :]
    ind = degt_ref[:, 1:2] + degt_ref[:, 3:4]
    od = degt_ref[:, 0:1] + degt_ref[:, 2:3]
    nd = lax.rsqrt(jnp.maximum(ind, 1.0))
    ns = lax.rsqrt(jnp.maximum(od, 1.0))
    h1 = jnp.dot(agg1 * nd, w1_ref[...], preferred_element_type=jnp.float32)
    h1 = jnp.maximum(h1 + b1_ref[...], 0.0)
    w2f = jnp.dot(w2_ref[...], wfc_ref[...], preferred_element_type=jnp.float32)
    g = jnp.dot(h1 * ns, w2f, preferred_element_type=jnp.float32)
    out_ref[0:N, :] = g
    out_ref[N:NPAD, :] = jnp.zeros((NPAD - N, NCLS), jnp.float32)


def _fin_body(agg2_ref, degt_ref, b2_ref, wfc_ref, bfc_ref, out_ref):
    agg2 = agg2_ref[0:N, :] + agg2_ref[NPAD:NPAD + N, :]
    ind = degt_ref[:, 1:2] + degt_ref[:, 3:4]
    nd = lax.rsqrt(jnp.maximum(ind, 1.0))
    bf = jnp.dot(b2_ref[...].reshape(1, NHID), wfc_ref[...],
                 preferred_element_type=jnp.float32)
    out_ref[...] = agg2 * nd + bf + bfc_ref[...].reshape(1, NCLS)


def kernel(edge_index, x, W1, b1, W2, b2, Wfc, bfc):
    ei = edge_index.astype(jnp.int32)
    pad = (jnp.arange(E_PAD - E, dtype=jnp.int32) % (NPAD - N)) + N
    src = jnp.concatenate([ei[0], pad]).reshape(W_TOT, K)
    dst = jnp.concatenate([ei[1], pad]).reshape(W_TOT, K)

    degs = _deg_kernel(src, dst)
    degt = jnp.transpose(degs.reshape(4, N))

    h = pl.pallas_call(
        _prep_body,
        out_shape=jax.ShapeDtypeStruct((NPAD, NFEAT), jnp.float32),
    )(x, degt)

    aggp = _agg128(h, src, dst)

    g = pl.pallas_call(
        _mid_body,
        out_shape=jax.ShapeDtypeStruct((NPAD, NCLS), jnp.float32),
    )(aggp, degt, W1, b1, W2, Wfc)

    agg2 = _agg16(g, src, dst)

    out = pl.pallas_call(
        _fin_body,
        out_shape=jax.ShapeDtypeStruct((N, NCLS), jnp.float32),
    )(agg2, degt, b2, Wfc, bfc)
    return out


# speedup vs baseline: 1.0831x; 1.0210x over previous
import functools

import jax
import jax.numpy as jnp
from jax import lax
from jax.experimental import pallas as pl
from jax.experimental.pallas import tpu as pltpu
from jax.experimental.pallas import tpu_sc as plsc

N = 10000
NPAD = 10240
E = 320000
K = 128
NC = 2
NS = 16
W_TOT = 2560
E_PAD = W_TOT * K
WPT = W_TOT // (NC * NS)
NFEAT = 128
NHID = 256
NCLS = 16
DEG_T = 10240
RPT = NPAD // NS

_MESH = dict(core_axis_name="c", subcore_axis_name="s")


@functools.partial(
    pl.kernel,
    out_type=jax.ShapeDtypeStruct((4 * N,), jnp.float32),
    mesh=plsc.VectorSubcoreMesh(**_MESH),
    compiler_params=pltpu.CompilerParams(use_tc_tiling_on_sc=False),
    scratch_types=[
        pltpu.VMEM((WPT, K), jnp.int32),
        pltpu.VMEM((WPT, K), jnp.int32),
        pltpu.VMEM((K,), jnp.float32),
        pltpu.VMEM((640,), jnp.float32),
        pltpu.VMEM((1000,), jnp.float32),
        pltpu.VMEM_SHARED((DEG_T,), jnp.float32),
        pltpu.VMEM_SHARED((DEG_T,), jnp.float32),
        pltpu.SemaphoreType.DMA,
    ],
)
def _deg_kernel(src_hbm, dst_hbm, out_hbm, src_v, dst_v, ones_v, z_v,
                stage_v, odeg_sh, ideg_sh, dsem):
    c = lax.axis_index("c")
    s = lax.axis_index("s")
    wid = c * NS + s

    def fill_ones(i, _):
        ones_v[pl.ds(i * 16, 16)] = jnp.full((16,), 1.0, jnp.float32)
        return 0

    lax.fori_loop(0, K // 16, fill_ones, 0)

    def fill_z(i, _):
        z_v[pl.ds(i * 16, 16)] = jnp.zeros((16,), jnp.float32)
        return 0

    lax.fori_loop(0, 640 // 16, fill_z, 0)

    pltpu.sync_copy(z_v, odeg_sh.at[pl.ds(s * 640, 640)])
    pltpu.sync_copy(z_v, ideg_sh.at[pl.ds(s * 640, 640)])
    pltpu.sync_copy(src_hbm.at[pl.ds(wid * WPT, WPT)], src_v)
    pltpu.sync_copy(dst_hbm.at[pl.ds(wid * WPT, WPT)], dst_v)
    plsc.subcore_barrier()

    def win(i, _):
        pltpu.async_copy(ones_v, odeg_sh.at[src_v.at[i]], dsem, add=True)
        pltpu.async_copy(ones_v, ideg_sh.at[dst_v.at[i]], dsem, add=True)
        return 0

    lax.fori_loop(0, WPT, win, 0)

    def drain(i, _):
        pltpu.make_async_copy(ones_v, odeg_sh.at[src_v.at[i]], dsem).wait()
        pltpu.make_async_copy(ones_v, ideg_sh.at[dst_v.at[i]], dsem).wait()
        return 0

    lax.fori_loop(0, WPT, drain, 0)
    plsc.subcore_barrier()

    @pl.when(s < 10)
    def _():
        base = s * 1000
        pltpu.sync_copy(odeg_sh.at[pl.ds(base, 1000)], stage_v)
        pltpu.sync_copy(stage_v,
                        out_hbm.at[pl.ds((2 * c + 0) * N + base, 1000)])
        pltpu.sync_copy(ideg_sh.at[pl.ds(base, 1000)], stage_v)
        pltpu.sync_copy(stage_v,
                        out_hbm.at[pl.ds((2 * c + 1) * N + base, 1000)])


RING = 6
PREF = 3


def _ring_pipeline(tab_hbm, src_v, dst_v, rows_v, acc_sh, gsem, ssem, nwin,
                   ring, pref, w0=0):
    for k in range(pref):
        pltpu.async_copy(tab_hbm.at[src_v.at[w0 + k]], rows_v.at[k],
                         gsem.at[k])

    def win(j, _):
        i = w0 + j
        bn = lax.rem(j + pref, ring)

        @pl.when(j + pref < nwin)
        def _():
            @pl.when(j >= ring - pref)
            def _():
                pltpu.make_async_copy(rows_v.at[bn],
                                      acc_sh.at[dst_v.at[i - ring + pref]],
                                      ssem.at[bn]).wait()

            pltpu.async_copy(tab_hbm.at[src_v.at[i + pref]], rows_v.at[bn],
                             gsem.at[bn])

        b = lax.rem(j, ring)
        pltpu.make_async_copy(tab_hbm.at[src_v.at[i]], rows_v.at[b],
                              gsem.at[b]).wait()
        pltpu.async_copy(rows_v.at[b], acc_sh.at[dst_v.at[i]],
                         ssem.at[b], add=True)
        return 0

    lax.fori_loop(0, nwin, win, 0)
    for k in range(ring):
        w = w0 + nwin - ring + k
        pltpu.make_async_copy(rows_v.at[(w - w0) % ring],
                              acc_sh.at[dst_v.at[w]],
                              ssem.at[(w - w0) % ring]).wait()


def _zero_acc(rows_v, acc_sh, s, D):
    def zrow(i, _):
        for j in range(D // 16):
            rows_v[0, i, pl.ds(j * 16, 16)] = jnp.zeros((16,), jnp.float32)
        return 0

    lax.fori_loop(0, K, zrow, 0)
    for r in range(RPT // K):
        pltpu.sync_copy(rows_v.at[0], acc_sh.at[pl.ds(s * RPT + r * K, K)])


DHALF = NFEAT // 2
WPH = WPT // 2


@functools.partial(
    pl.kernel,
    out_type=jax.ShapeDtypeStruct((2 * NPAD, NFEAT), jnp.float32),
    mesh=plsc.VectorSubcoreMesh(**_MESH),
    scratch_types=[
        pltpu.VMEM((WPH, K), jnp.int32),
        pltpu.VMEM((WPH, K), jnp.int32),
        pltpu.VMEM((2, K, NFEAT), jnp.float32),
        pltpu.VMEM_SHARED((NPAD, NFEAT), jnp.float32),
        pltpu.SemaphoreType.DMA((2,)),
        pltpu.SemaphoreType.DMA((2,)),
    ],
)
def _agg128(tab_hbm, src_hbm, dst_hbm, out_hbm,
            src_v, dst_v, rows_v, acc_sh, gsem, ssem):
    c = lax.axis_index("c")
    s = lax.axis_index("s")
    wid = c * NS + s

    _zero_acc(rows_v, acc_sh, s, NFEAT)
    plsc.subcore_barrier()

    for ph in range(2):
        base_w = wid * WPT + ph * WPH
        pltpu.sync_copy(src_hbm.at[pl.ds(base_w, WPH)], src_v)
        pltpu.sync_copy(dst_hbm.at[pl.ds(base_w, WPH)], dst_v)
        _ring_pipeline(tab_hbm, src_v, dst_v, rows_v, acc_sh, gsem, ssem,
                       WPH, ring=2, pref=1)

    plsc.subcore_barrier()
    for r in range(RPT // K):
        pltpu.sync_copy(acc_sh.at[pl.ds(s * RPT + r * K, K)], rows_v.at[0])
        pltpu.sync_copy(rows_v.at[0],
                        out_hbm.at[pl.ds(c * NPAD + s * RPT + r * K, K)])


@functools.partial(
    pl.kernel,
    out_type=jax.ShapeDtypeStruct((2 * NPAD, NCLS), jnp.float32),
    mesh=plsc.VectorSubcoreMesh(**_MESH),
    compiler_params=pltpu.CompilerParams(use_tc_tiling_on_sc=False),
    scratch_types=[
        pltpu.VMEM((WPT, K), jnp.int32),
        pltpu.VMEM((WPT, K), jnp.int32),
        pltpu.VMEM((RING, K, NCLS), jnp.float32),
        pltpu.VMEM_SHARED((NPAD, NCLS), jnp.float32),
        pltpu.SemaphoreType.DMA((RING,)),
        pltpu.SemaphoreType.DMA((RING,)),
    ],
)
def _agg16(tab_hbm, src_hbm, dst_hbm, out_hbm,
           src_v, dst_v, rows_v, acc_sh, gsem, ssem):
    c = lax.axis_index("c")
    s = lax.axis_index("s")
    wid = c * NS + s

    _zero_acc(rows_v, acc_sh, s, NCLS)
    pltpu.sync_copy(src_hbm.at[pl.ds(wid * WPT, WPT)], src_v)
    pltpu.sync_copy(dst_hbm.at[pl.ds(wid * WPT, WPT)], dst_v)
    plsc.subcore_barrier()

    _ring_pipeline(tab_hbm, src_v, dst_v, rows_v, acc_sh, gsem, ssem, WPT,
                   ring=RING, pref=PREF)

    plsc.subcore_barrier()
    for r in range(RPT // K):
        pltpu.sync_copy(acc_sh.at[pl.ds(s * RPT + r * K, K)], rows_v.at[0])
        pltpu.sync_copy(rows_v.at[0],
                        out_hbm.at[pl.ds(c * NPAD + s * RPT + r * K, K)])


def _prep_body(x_ref, degt_ref, out_ref):
    od = degt_ref[:, 0:1] + degt_ref[:, 2:3]
    ns = lax.rsqrt(jnp.maximum(od, 1.0))
    out_ref[0:N, :] = x_ref[...] * ns
    out_ref[N:NPAD, :] = jnp.zeros((NPAD - N, NFEAT), jnp.float32)


def _mid_body(aggp_ref, degt_ref, w1_ref, b1_ref, w2_ref, wfc_ref, out_ref):
    agg1 = aggp_ref[0:N, :] + aggp_ref[NPAD:NPAD + N, :]
    ind = degt_ref[:, 1:2] + degt_ref[:, 3:4]
    od = degt_ref[:, 0:1] + degt_ref[:, 2:3]
    nd = lax.rsqrt(jnp.maximum(ind, 1.0))
    ns = lax.rsqrt(jnp.maximum(od, 1.0))
    h1 = jnp.dot(agg1 * nd, w1_ref[...], preferred_element_type=jnp.float32)
    h1 = jnp.maximum(h1 + b1_ref[...], 0.0)
    w2f = jnp.dot(w2_ref[...], wfc_ref[...], preferred_element_type=jnp.float32)
    g = jnp.dot(h1 * ns, w2f, preferred_element_type=jnp.float32)
    out_ref[0:N, :] = g
    out_ref[N:NPAD, :] = jnp.zeros((NPAD - N, NCLS), jnp.float32)


def _fin_body(agg2_ref, degt_ref, b2_ref, wfc_ref, bfc_ref, out_ref):
    agg2 = agg2_ref[0:N, :] + agg2_ref[NPAD:NPAD + N, :]
    ind = degt_ref[:, 1:2] + degt_ref[:, 3:4]
    nd = lax.rsqrt(jnp.maximum(ind, 1.0))
    bf = jnp.dot(b2_ref[...].reshape(1, NHID), wfc_ref[...],
                 preferred_element_type=jnp.float32)
    out_ref[...] = agg2 * nd + bf + bfc_ref[...].reshape(1, NCLS)


def kernel(edge_index, x, W1, b1, W2, b2, Wfc, bfc):
    ei = edge_index.astype(jnp.int32)
    pad = (jnp.arange(E_PAD - E, dtype=jnp.int32) % (NPAD - N)) + N
    src = jnp.concatenate([ei[0], pad]).reshape(W_TOT, K)
    dst = jnp.concatenate([ei[1], pad]).reshape(W_TOT, K)

    degs = _deg_kernel(src, dst)
    degt = jnp.transpose(degs.reshape(4, N))

    h = pl.pallas_call(
        _prep_body,
        out_shape=jax.ShapeDtypeStruct((NPAD, NFEAT), jnp.float32),
    )(x, degt)

    aggp = _agg128(h, src, dst)

    g = pl.pallas_call(
        _mid_body,
        out_shape=jax.ShapeDtypeStruct((NPAD, NCLS), jnp.float32),
    )(aggp, degt, W1, b1, W2, Wfc)

    agg2 = _agg16(g, src, dst)

    out = pl.pallas_call(
        _fin_body,
        out_shape=jax.ShapeDtypeStruct((N, NCLS), jnp.float32),
    )(agg2, degt, b2, Wfc, bfc)
    return out
